# initial kernel scaffold (unmeasured)
import jax
import jax.numpy as jnp
from jax import lax
from jax.experimental import pallas as pl
from jax.experimental.pallas import tpu as pltpu


def kernel(Q, K, V):
    b, sq, h, d = Q.shape
    skv = K.shape[1]
    scale = d ** -0.5
    pack_w = 128

    def body(q_ref, k_ref, v_ref, o_ref, send_buf, recv_buf, send_sem, recv_sem):
        my_x = lax.axis_index("x")
        my_y = lax.axis_index("y")
        my_z = lax.axis_index("z")
        peer = (1 - my_x, my_y, my_z)

        barrier = pltpu.get_barrier_semaphore()
        pl.semaphore_signal(
            barrier, inc=1, device_id=peer, device_id_type=pl.DeviceIdType.MESH
        )
        pl.semaphore_wait(barrier, 1)

        q = q_ref[...].reshape(b, h, d)
        kt = jnp.transpose(k_ref[...], (0, 2, 1, 3))
        vt = jnp.transpose(v_ref[...], (0, 2, 1, 3))
        s = jnp.sum(q[:, :, None, :] * kt, axis=-1) * scale
        m = jnp.max(s, axis=-1, keepdims=True)
        p = jnp.exp(s - m)
        l = jnp.sum(p, axis=-1, keepdims=True)
        o = jnp.sum(p[..., None] * vt, axis=2)

        send_buf[:, :, 0:d] = o
        send_buf[:, :, d:d + 1] = m
        send_buf[:, :, d + 1:d + 2] = l

        rdma = pltpu.make_async_remote_copy(
            src_ref=send_buf,
            dst_ref=recv_buf,
            send_sem=send_sem,
            recv_sem=recv_sem,
            device_id=peer,
            device_id_type=pl.DeviceIdType.MESH,
        )
        rdma.start()
        rdma.wait()

        po = recv_buf[:, :, 0:d]
        pm = recv_buf[:, :, d:d + 1]
        pl_ = recv_buf[:, :, d + 1:d + 2]

        mx = jnp.maximum(m, pm)
        a = jnp.exp(m - mx)
        c = jnp.exp(pm - mx)
        merged = (o * a + po * c) / (l * a + pl_ * c)
        o_ref[...] = merged.reshape(b, sq, h, d)

    return pl.pallas_call(
        body,
        out_shape=jax.ShapeDtypeStruct((b, sq, h, d), jnp.float32),
        in_specs=[
            pl.BlockSpec(memory_space=pltpu.VMEM),
            pl.BlockSpec(memory_space=pltpu.VMEM),
            pl.BlockSpec(memory_space=pltpu.VMEM),
        ],
        out_specs=pl.BlockSpec(memory_space=pltpu.VMEM),
        scratch_shapes=[
            pltpu.VMEM((b, h, pack_w), jnp.float32),
            pltpu.VMEM((b, h, pack_w), jnp.float32),
            pltpu.SemaphoreType.DMA,
            pltpu.SemaphoreType.DMA,
        ],
        compiler_params=pltpu.CompilerParams(collective_id=0),
    )(Q, K, V)


# baseline (device time: 54397 ns/iter reference)
import jax
import jax.numpy as jnp
from jax import lax
from jax.experimental import pallas as pl
from jax.experimental.pallas import tpu as pltpu


def kernel(Q, K, V):
    b, sq, h, d = Q.shape
    skv = K.shape[1]
    scale = d ** -0.5
    pack_w = 128

    def body(q_ref, k_ref, v_ref, o_ref, send_buf, recv_buf, send_sem, recv_sem):
        my_x = lax.axis_index("x")
        my_y = lax.axis_index("y")
        my_z = lax.axis_index("z")
        peer = (1 - my_x, my_y, my_z)

        barrier = pltpu.get_barrier_semaphore()
        pl.semaphore_signal(
            barrier, inc=1, device_id=peer, device_id_type=pl.DeviceIdType.MESH
        )
        pl.semaphore_wait(barrier, 1)

        q = q_ref[...].reshape(b, h, d).astype(jnp.bfloat16)
        kt = jnp.transpose(k_ref[...].astype(jnp.bfloat16), (0, 2, 1, 3))
        vt = jnp.transpose(v_ref[...].astype(jnp.bfloat16), (0, 2, 1, 3))
        s = jnp.sum(
            q[:, :, None, :] * kt, axis=-1, dtype=jnp.float32
        ) * scale
        m = jnp.max(s, axis=-1, keepdims=True)
        p = jnp.exp(s - m).astype(jnp.bfloat16)
        l = jnp.sum(p, axis=-1, keepdims=True, dtype=jnp.float32)
        o = jnp.sum(p[..., None] * vt, axis=2, dtype=jnp.float32)

        send_buf[:, :, 0:d] = o
        send_buf[:, :, d:d + 1] = m
        send_buf[:, :, d + 1:d + 2] = l

        rdma = pltpu.make_async_remote_copy(
            src_ref=send_buf,
            dst_ref=recv_buf,
            send_sem=send_sem,
            recv_sem=recv_sem,
            device_id=peer,
            device_id_type=pl.DeviceIdType.MESH,
        )
        rdma.start()
        rdma.wait()

        po = recv_buf[:, :, 0:d]
        pm = recv_buf[:, :, d:d + 1]
        pl_ = recv_buf[:, :, d + 1:d + 2]

        mx = jnp.maximum(m, pm)
        a = jnp.exp(m - mx)
        c = jnp.exp(pm - mx)
        merged = (o * a + po * c) / (l * a + pl_ * c)
        o_ref[...] = merged.reshape(b, sq, h, d)

    return pl.pallas_call(
        body,
        out_shape=jax.ShapeDtypeStruct((b, sq, h, d), jnp.float32),
        in_specs=[
            pl.BlockSpec(memory_space=pltpu.VMEM),
            pl.BlockSpec(memory_space=pltpu.VMEM),
            pl.BlockSpec(memory_space=pltpu.VMEM),
        ],
        out_specs=pl.BlockSpec(memory_space=pltpu.VMEM),
        scratch_shapes=[
            pltpu.VMEM((b, h, pack_w), jnp.float32),
            pltpu.VMEM((b, h, pack_w), jnp.float32),
            pltpu.SemaphoreType.DMA,
            pltpu.SemaphoreType.DMA,
        ],
        compiler_params=pltpu.CompilerParams(
            collective_id=0, vmem_limit_bytes=100 * 1024 * 1024
        ),
    )(Q, K, V)


# device time: 53189 ns/iter; 1.0227x vs baseline; 1.0227x over previous
import jax
import jax.numpy as jnp
from jax import lax
from jax.experimental import pallas as pl
from jax.experimental.pallas import tpu as pltpu


def kernel(Q, K, V):
    b, sq, h, d = Q.shape
    skv = K.shape[1]
    scale = d ** -0.5
    pack_w = 128

    def body(q_ref, k_ref, v_ref, o_ref, send_buf, recv_buf, send_sem, recv_sem):
        my_x = lax.axis_index("x")
        my_y = lax.axis_index("y")
        my_z = lax.axis_index("z")
        peer = (1 - my_x, my_y, my_z)

        barrier = pltpu.get_barrier_semaphore()
        pl.semaphore_signal(
            barrier, inc=1, device_id=peer, device_id_type=pl.DeviceIdType.MESH
        )
        pl.semaphore_wait(barrier, 1)

        q4 = q_ref[...].astype(jnp.bfloat16)
        kb = k_ref[...].astype(jnp.bfloat16)
        vb = v_ref[...].astype(jnp.bfloat16)
        s = jnp.sum(q4 * kb, axis=-1, dtype=jnp.float32) * scale
        ms = jnp.max(s, axis=1, keepdims=True)
        ps = jnp.exp(s - ms).astype(jnp.bfloat16)
        ls = jnp.sum(ps, axis=1, keepdims=True, dtype=jnp.float32)
        o = jnp.sum(ps[..., None] * vb, axis=1, dtype=jnp.float32)
        m = jnp.transpose(ms, (0, 2, 1))
        l = jnp.transpose(ls, (0, 2, 1))

        send_buf[:, :, 0:d] = o
        send_buf[:, :, d:d + 1] = m
        send_buf[:, :, d + 1:d + 2] = l

        rdma = pltpu.make_async_remote_copy(
            src_ref=send_buf,
            dst_ref=recv_buf,
            send_sem=send_sem,
            recv_sem=recv_sem,
            device_id=peer,
            device_id_type=pl.DeviceIdType.MESH,
        )
        rdma.start()
        rdma.wait()

        po = recv_buf[:, :, 0:d]
        pm = recv_buf[:, :, d:d + 1]
        pl_ = recv_buf[:, :, d + 1:d + 2]

        mx = jnp.maximum(m, pm)
        a = jnp.exp(m - mx)
        c = jnp.exp(pm - mx)
        merged = (o * a + po * c) / (l * a + pl_ * c)
        o_ref[...] = merged.reshape(b, sq, h, d)

    return pl.pallas_call(
        body,
        out_shape=jax.ShapeDtypeStruct((b, sq, h, d), jnp.float32),
        in_specs=[
            pl.BlockSpec(memory_space=pltpu.VMEM),
            pl.BlockSpec(memory_space=pltpu.VMEM),
            pl.BlockSpec(memory_space=pltpu.VMEM),
        ],
        out_specs=pl.BlockSpec(memory_space=pltpu.VMEM),
        scratch_shapes=[
            pltpu.VMEM((b, h, pack_w), jnp.float32),
            pltpu.VMEM((b, h, pack_w), jnp.float32),
            pltpu.SemaphoreType.DMA,
            pltpu.SemaphoreType.DMA,
        ],
        compiler_params=pltpu.CompilerParams(
            collective_id=0, vmem_limit_bytes=100 * 1024 * 1024
        ),
    )(Q, K, V)


# device time: 51504 ns/iter; 1.0562x vs baseline; 1.0327x over previous
import jax
import jax.numpy as jnp
from jax import lax
from jax.experimental import pallas as pl
from jax.experimental.pallas import tpu as pltpu


def kernel(Q, K, V):
    b, sq, h, d = Q.shape
    skv = K.shape[1]
    scale = d ** -0.5
    pack_w = 128

    def body(q_ref, k_ref, v_ref, o_ref, send_buf, recv_buf, send_sem, recv_sem):
        my_x = lax.axis_index("x")
        my_y = lax.axis_index("y")
        my_z = lax.axis_index("z")
        peer = (1 - my_x, my_y, my_z)

        barrier = pltpu.get_barrier_semaphore()
        pl.semaphore_signal(
            barrier, inc=1, device_id=peer, device_id_type=pl.DeviceIdType.MESH
        )
        pl.semaphore_wait(barrier, 1)

        row = lax.broadcasted_iota(jnp.int32, (h, h), 0)
        col = lax.broadcasted_iota(jnp.int32, (h, h), 1)
        eye = (row == col).astype(jnp.float32)

        with jax.named_scope("convert"):
            qs = (q_ref[...].reshape(b, h, d) * scale).astype(jnp.bfloat16)
            kb = k_ref[...].astype(jnp.bfloat16).reshape(b, skv * h, d)
            vb = v_ref[...].astype(jnp.bfloat16).reshape(b, skv, h * d)
        with jax.named_scope("scores"):
            s_full = lax.dot_general(
                kb, qs, (((2,), (2,)), ((0,), (0,))),
                preferred_element_type=jnp.float32,
            )
            s3 = s_full.reshape(b, skv, h, h)
            s = jnp.sum(s3 * eye[None, None], axis=2)
        with jax.named_scope("softmax"):
            ms = jnp.max(s, axis=1, keepdims=True)
            ps = jnp.exp(s - ms)
            ls = jnp.sum(ps, axis=1, keepdims=True, dtype=jnp.float32)
        with jax.named_scope("pv"):
            pt = jnp.transpose(ps, (0, 2, 1)).astype(jnp.bfloat16)
            o_full = lax.dot_general(
                pt, vb, (((2,), (1,)), ((0,), (0,))),
                preferred_element_type=jnp.float32,
            )
            o4 = o_full.reshape(b, h, h, d)
            o = jnp.sum(o4 * eye[None, :, :, None], axis=2)
        with jax.named_scope("pack_send"):
            m = jnp.transpose(ms, (0, 2, 1))
            l = jnp.transpose(ls, (0, 2, 1))
            send_buf[:, :, 0:d] = o
            send_buf[:, :, d:d + 1] = m
            send_buf[:, :, d + 1:d + 2] = l
            rdma = pltpu.make_async_remote_copy(
                src_ref=send_buf,
                dst_ref=recv_buf,
                send_sem=send_sem,
                recv_sem=recv_sem,
                device_id=peer,
                device_id_type=pl.DeviceIdType.MESH,
            )
            rdma.start()
        with jax.named_scope("rdma_wait"):
            rdma.wait()
        with jax.named_scope("merge"):
            po = recv_buf[:, :, 0:d]
            pm = recv_buf[:, :, d:d + 1]
            pl_ = recv_buf[:, :, d + 1:d + 2]
            mx = jnp.maximum(m, pm)
            a = jnp.exp(m - mx)
            c = jnp.exp(pm - mx)
            merged = (o * a + po * c) / (l * a + pl_ * c)
            o_ref[...] = merged.reshape(b, sq, h, d)

    return pl.pallas_call(
        body,
        out_shape=jax.ShapeDtypeStruct((b, sq, h, d), jnp.float32),
        in_specs=[
            pl.BlockSpec(memory_space=pltpu.VMEM),
            pl.BlockSpec(memory_space=pltpu.VMEM),
            pl.BlockSpec(memory_space=pltpu.VMEM),
        ],
        out_specs=pl.BlockSpec(memory_space=pltpu.VMEM),
        scratch_shapes=[
            pltpu.VMEM((b, h, pack_w), jnp.float32),
            pltpu.VMEM((b, h, pack_w), jnp.float32),
            pltpu.SemaphoreType.DMA,
            pltpu.SemaphoreType.DMA,
        ],
        compiler_params=pltpu.CompilerParams(
            collective_id=0, vmem_limit_bytes=100 * 1024 * 1024
        ),
    )(Q, K, V)
